# Initial kernel scaffold; baseline (speedup 1.0000x reference)
import jax
import jax.numpy as jnp
from jax.experimental import pallas as pl


def kernel(potential, sheet_thickness, channel_size, bedrock_elevation, ice_thickness, sliding_velocity, link_length, node_at_link_head, node_at_link_tail, status_at_node):
    raise NotImplementedError("write your pallas kernel here")


# speedup vs baseline: 318.8758x; 318.8758x over previous
import jax
import jax.numpy as jnp
from jax import lax
from jax.experimental import pallas as pl
from jax.experimental.pallas import tpu as pltpu
from jax.experimental.pallas import tpu_sc as plsc

N_NODES = 50000
N_LINKS = 1600000
NPAD = 50176
CHUNK = 2560
NCHUNKS = N_LINKS // CHUNK
NW = 32
STRIPE = NPAD // 16

WATER_DENSITY = 1000.0
ICE_DENSITY = 917.0
GRAVITY = 9.81
SEC_PER_A = 31556926.0
SHEET_CONDUCTIVITY = 0.01
CHANNEL_CONDUCTIVITY = 0.1
BEDROCK_STEP_HEIGHT = 0.1
CAVITY_SPACING = 2.0
CLOSURE_COEFF = 5e-25
HEAT_COEFF = -(7.5e-08 * 4220.0 * 1000.0)

f32 = jnp.float32
i32 = jnp.int32

_mesh = plsc.VectorSubcoreMesh(
    core_axis_name="c", subcore_axis_name="s", num_cores=2, num_subcores=16)

_sc_params = pltpu.CompilerParams(needs_layout_passes=False)


def _rsqrt(x):
    bits = plsc.bitcast(x, i32)
    y = plsc.bitcast(jnp.int32(0x5F3759DF) - lax.shift_right_arithmetic(bits, 1), f32)
    for _ in range(3):
        y = y * (1.5 - 0.5 * x * y * y)
    return y


def _phase1_body(head_hbm, tail_hbm, sl_hbm, bed_hbm,
                 dpart_hbm, spart_hbm, dbase_hbm,
                 base_v, headb, tailb, slb, dvh, dvt, svb, dbb, zb,
                 dsh, ssh):
    cid = lax.axis_index("c")
    sid = lax.axis_index("s")
    wid = sid * 2 + cid

    pltpu.sync_copy(bed_hbm, base_v)

    def scale(i, carry):
        sl_ = pl.ds(i * 16, 16)
        base_v[sl_] = base_v[sl_] * (WATER_DENSITY * GRAVITY)
        return carry
    lax.fori_loop(0, NPAD // 16, scale, 0)

    def zero(i, carry):
        zb[pl.ds(i * 16, 16)] = jnp.zeros((16,), f32)
        return carry
    lax.fori_loop(0, STRIPE // 16, zero, 0)
    pltpu.sync_copy(zb, dsh.at[pl.ds(sid * STRIPE, STRIPE)])
    pltpu.sync_copy(zb, ssh.at[pl.ds(sid * STRIPE, STRIPE)])
    plsc.subcore_barrier()

    trip = (NCHUNKS - wid + NW - 1) // NW

    def chunk(t, carry):
        l0 = (wid + t * NW) * CHUNK
        pltpu.sync_copy(head_hbm.at[pl.ds(l0, CHUNK)], headb)
        pltpu.sync_copy(tail_hbm.at[pl.ds(l0, CHUNK)], tailb)
        pltpu.sync_copy(sl_hbm.at[pl.ds(l0, CHUNK)], slb)

        def step(i, c2):
            cs_ = pl.ds(i * 16, 16)
            hh = headb[cs_]
            tt = tailb[cs_]
            slv = slb[cs_]
            bh = plsc.load_gather(base_v, [hh])
            bt = plsc.load_gather(base_v, [tt])
            dvh[cs_] = jnp.where(bt < bh, f32(4097.0), f32(1.0))
            dvt[cs_] = jnp.where(bh < bt, f32(4097.0), f32(1.0))
            svb[cs_] = jnp.abs(slv) * (1.0 / SEC_PER_A)
            dbb[cs_] = bh - bt
            return c2
        lax.fori_loop(0, CHUNK // 16, step, 0)

        pltpu.sync_copy(dbb, dbase_hbm.at[pl.ds(l0, CHUNK)])
        pltpu.sync_copy(dvh, dsh.at[headb], add=True)
        pltpu.sync_copy(dvt, dsh.at[tailb], add=True)
        pltpu.sync_copy(svb, ssh.at[headb], add=True)
        pltpu.sync_copy(svb, ssh.at[tailb], add=True)
        return carry
    lax.fori_loop(0, trip, chunk, 0)

    plsc.subcore_barrier()

    @pl.when(sid == 0)
    def _():
        pltpu.sync_copy(dsh, dpart_hbm.at[cid])
        pltpu.sync_copy(ssh, spart_hbm.at[cid])


_phase1 = pl.kernel(
    _phase1_body,
    out_type=(
        jax.ShapeDtypeStruct((2, NPAD), f32),
        jax.ShapeDtypeStruct((2, NPAD), f32),
        jax.ShapeDtypeStruct((N_LINKS,), f32),
    ),
    mesh=_mesh,
    scratch_types=[
        pltpu.VMEM((NPAD,), f32),
        pltpu.VMEM((CHUNK,), i32),
        pltpu.VMEM((CHUNK,), i32),
        pltpu.VMEM((CHUNK,), f32),
        pltpu.VMEM((CHUNK,), f32),
        pltpu.VMEM((CHUNK,), f32),
        pltpu.VMEM((CHUNK,), f32),
        pltpu.VMEM((CHUNK,), f32),
        pltpu.VMEM((STRIPE,), f32),
        pltpu.VMEM_SHARED((NPAD,), f32),
        pltpu.VMEM_SHARED((NPAD,), f32),
    ],
    compiler_params=_sc_params,
)


def _phase2_body(d_ref, s_ref, p_ref, sh_ref, bed_ref, ice_ref, st_ref,
                 open_ref, scl_ref, ptag_ref, bpk_ref):
    D = d_ref[0] + d_ref[1]
    S = s_ref[0] + s_ref[1]
    inds = jnp.floor(D * (1.0 / 4096.0))
    counts = D - 4096.0 * inds
    p = p_ref[...]
    s = sh_ref[...]
    tag = jnp.logical_and(st_ref[...] > 0, D >= 4096.0)
    sliding_node = S / jnp.maximum(counts, 1.0)
    open_ref[...] = jnp.where(
        s < BEDROCK_STEP_HEIGHT,
        sliding_node * (BEDROCK_STEP_HEIGHT - s) * (1.0 / CAVITY_SPACING), 0.0)
    base = f32(WATER_DENSITY * GRAVITY) * bed_ref[...]
    ovb = base + f32(ICE_DENSITY * GRAVITY) * ice_ref[...]
    neff = ovb - p
    rn = jnp.maximum(neff, 0.0)
    scl_ref[...] = f32(CLOSURE_COEFF) * s * (rn * rn * rn)
    pbits = lax.bitcast_convert_type(p, i32)
    ptag_ref[...] = lax.bitcast_convert_type(
        (pbits & jnp.int32(-2)) | tag.astype(i32), f32)
    s16 = lax.bitcast_convert_type(s.astype(jnp.bfloat16), jnp.uint16).astype(i32)
    n16 = lax.bitcast_convert_type(neff.astype(jnp.bfloat16), jnp.uint16).astype(i32)
    bpk_ref[...] = lax.shift_left(n16, 16) | s16


_NSHAPE = (NPAD // 128, 128)


def _phase2(dpart, spart, p2, s2, bed2, ice2, st2):
    return pl.pallas_call(
        _phase2_body,
        out_shape=(
            jax.ShapeDtypeStruct(_NSHAPE, f32),
            jax.ShapeDtypeStruct(_NSHAPE, f32),
            jax.ShapeDtypeStruct(_NSHAPE, f32),
            jax.ShapeDtypeStruct(_NSHAPE, i32),
        ),
    )(dpart, spart, p2, s2, bed2, ice2, st2)


def _phase3_body(head_hbm, tail_hbm, cs_hbm, len_hbm, db_hbm, ptag_hbm, bpk_hbm,
                 diss_hbm, sens_hbm, ccl_hbm,
                 ptag_v, bpk_v, headb, tailb, csb, lenb, dbb, dob, sob, cob):
    cid = lax.axis_index("c")
    sid = lax.axis_index("s")
    wid = sid * 2 + cid

    pltpu.sync_copy(ptag_hbm, ptag_v)
    pltpu.sync_copy(bpk_hbm, bpk_v)

    trip = (NCHUNKS - wid + NW - 1) // NW

    def chunk(t, carry):
        l0 = (wid + t * NW) * CHUNK
        pltpu.sync_copy(head_hbm.at[pl.ds(l0, CHUNK)], headb)
        pltpu.sync_copy(tail_hbm.at[pl.ds(l0, CHUNK)], tailb)
        pltpu.sync_copy(cs_hbm.at[pl.ds(l0, CHUNK)], csb)
        pltpu.sync_copy(len_hbm.at[pl.ds(l0, CHUNK)], lenb)
        pltpu.sync_copy(db_hbm.at[pl.ds(l0, CHUNK)], dbb)

        def step(i, c2):
            cs_ = pl.ds(i * 16, 16)
            hh = headb[cs_]
            tt = tailb[cs_]
            ph = plsc.load_gather(ptag_v, [hh])
            pt = plsc.load_gather(ptag_v, [tt])
            bh = plsc.load_gather(bpk_v, [hh])
            bt = plsc.load_gather(bpk_v, [tt])
            phb = plsc.bitcast(ph, i32)
            ptb = plsc.bitcast(pt, i32)
            okm = ((phb | ptb) & 1) == 0
            lenv = lenb[cs_]
            csv = csb[cs_]
            dbv = dbb[cs_]
            rl = 1.0 / lenv
            dp = ph - pt
            g = jnp.where(okm, dp * rl, f32(0.0))
            s_h = plsc.bitcast(lax.shift_left(bh, 16), f32)
            s_t = plsc.bitcast(lax.shift_left(bt, 16), f32)
            ne_h = plsc.bitcast(bh & jnp.int32(-65536), f32)
            ne_t = plsc.bitcast(bt & jnp.int32(-65536), f32)
            h = 0.5 * (s_h + s_t)
            absg = jnp.abs(g)
            rg = _rsqrt(absg)
            rh = _rsqrt(h)
            sqh = h * rh
            rq = _rsqrt(sqh)
            h125 = h * (sqh * rq)
            gz = g == 0.0
            sheet_q = jnp.where(gz, f32(0.0), (-SHEET_CONDUCTIVITY) * h125 * rg) * g
            chan_q = jnp.where(gz, f32(0.0), (-CHANNEL_CONDUCTIVITY) * (csv * csv * csv)) * g
            dob[cs_] = jnp.abs(CAVITY_SPACING * sheet_q * g) + jnp.abs(chan_q * g)
            pgrad = jnp.where(okm, (dp - dbv) * rl, f32(0.0))
            tq = jnp.where((csv > 0.0) | (pgrad * sheet_q > 0.0),
                           chan_q + CAVITY_SPACING, chan_q)
            sob[cs_] = HEAT_COEFF * tq * pgrad
            nl = jnp.maximum(0.5 * (ne_h + ne_t), 0.0)
            cob[cs_] = f32(CLOSURE_COEFF) * csv * (nl * nl * nl)
            return c2
        lax.fori_loop(0, CHUNK // 16, step, 0)

        pltpu.sync_copy(dob, diss_hbm.at[pl.ds(l0, CHUNK)])
        pltpu.sync_copy(sob, sens_hbm.at[pl.ds(l0, CHUNK)])
        pltpu.sync_copy(cob, ccl_hbm.at[pl.ds(l0, CHUNK)])
        return carry
    lax.fori_loop(0, trip, chunk, 0)


_phase3 = pl.kernel(
    _phase3_body,
    out_type=(
        jax.ShapeDtypeStruct((N_LINKS,), f32),
        jax.ShapeDtypeStruct((N_LINKS,), f32),
        jax.ShapeDtypeStruct((N_LINKS,), f32),
    ),
    mesh=_mesh,
    scratch_types=[
        pltpu.VMEM((NPAD,), f32),
        pltpu.VMEM((NPAD,), i32),
        pltpu.VMEM((CHUNK,), i32),
        pltpu.VMEM((CHUNK,), i32),
        pltpu.VMEM((CHUNK,), f32),
        pltpu.VMEM((CHUNK,), f32),
        pltpu.VMEM((CHUNK,), f32),
        pltpu.VMEM((CHUNK,), f32),
        pltpu.VMEM((CHUNK,), f32),
        pltpu.VMEM((CHUNK,), f32),
    ],
    compiler_params=_sc_params,
)


def kernel(potential, sheet_thickness, channel_size, bedrock_elevation,
           ice_thickness, sliding_velocity, link_length,
           node_at_link_head, node_at_link_tail, status_at_node):
    head = node_at_link_head.astype(i32)
    tail = node_at_link_tail.astype(i32)

    pad = NPAD - N_NODES
    bedp = jnp.pad(bedrock_elevation, (0, pad))
    p2 = jnp.pad(potential, (0, pad)).reshape(_NSHAPE)
    s2 = jnp.pad(sheet_thickness, (0, pad)).reshape(_NSHAPE)
    bed2 = bedp.reshape(_NSHAPE)
    ice2 = jnp.pad(ice_thickness, (0, pad)).reshape(_NSHAPE)
    st2 = jnp.pad(status_at_node.astype(i32), (0, pad)).reshape(_NSHAPE)

    dpart, spart, dbase = _phase1(head, tail, sliding_velocity, bedp)

    dpart3 = dpart.reshape(2, NPAD // 128, 128)
    spart3 = spart.reshape(2, NPAD // 128, 128)
    opening, sheet_closure, ptag, bpk = _phase2(
        dpart3, spart3, p2, s2, bed2, ice2, st2)

    diss, sens, ccl = _phase3(
        head, tail, channel_size, link_length, dbase,
        ptag.reshape(NPAD), bpk.reshape(NPAD))

    return (diss, sens,
            opening.reshape(NPAD)[:N_NODES],
            sheet_closure.reshape(NPAD)[:N_NODES],
            ccl)


# speedup vs baseline: 414.5375x; 1.3000x over previous
import jax
import jax.numpy as jnp
from jax import lax
from jax.experimental import pallas as pl
from jax.experimental.pallas import tpu as pltpu
from jax.experimental.pallas import tpu_sc as plsc

N_NODES = 50000
N_LINKS = 1600000
NPAD = 50176
CHUNK = 2560
NCHUNKS = N_LINKS // CHUNK
NW = 32
STRIPE = NPAD // 16

WATER_DENSITY = 1000.0
ICE_DENSITY = 917.0
GRAVITY = 9.81
SEC_PER_A = 31556926.0
SHEET_CONDUCTIVITY = 0.01
CHANNEL_CONDUCTIVITY = 0.1
BEDROCK_STEP_HEIGHT = 0.1
CAVITY_SPACING = 2.0
CLOSURE_COEFF = 5e-25
HEAT_COEFF = -(7.5e-08 * 4220.0 * 1000.0)

f32 = jnp.float32
i32 = jnp.int32

_mesh = plsc.VectorSubcoreMesh(
    core_axis_name="c", subcore_axis_name="s", num_cores=2, num_subcores=16)

_sc_params = pltpu.CompilerParams(needs_layout_passes=False)


def _rsqrt(x):
    bits = plsc.bitcast(x, i32)
    y = plsc.bitcast(jnp.int32(0x5F3759DF) - lax.shift_right_arithmetic(bits, 1), f32)
    for _ in range(3):
        y = y * (1.5 - 0.5 * x * y * y)
    return y


def _rsqrt2(x):
    bits = plsc.bitcast(x, i32)
    y = plsc.bitcast(jnp.int32(0x5F3759DF) - lax.shift_right_arithmetic(bits, 1), f32)
    for _ in range(2):
        y = y * (1.5 - 0.5 * x * y * y)
    return y


def _phase1_body(head_hbm, tail_hbm, sl_hbm, bed_hbm,
                 dpart_hbm, spart_hbm, dbase_hbm,
                 base_v, headb, tailb, slb, dvh, dvt, svb, dbb, zb,
                 dsh, ssh, insem, outsem):
    cid = lax.axis_index("c")
    sid = lax.axis_index("s")
    wid = sid * 2 + cid

    pltpu.sync_copy(bed_hbm, base_v)

    def scale(i, carry):
        sl_ = pl.ds(i * 16, 16)
        base_v[sl_] = base_v[sl_] * (WATER_DENSITY * GRAVITY)
        return carry
    lax.fori_loop(0, NPAD // 16, scale, 0)

    def zero(i, carry):
        zb[pl.ds(i * 16, 16)] = jnp.zeros((16,), f32)
        return carry
    lax.fori_loop(0, STRIPE // 16, zero, 0)
    pltpu.sync_copy(zb, dsh.at[pl.ds(sid * STRIPE, STRIPE)])
    pltpu.sync_copy(zb, ssh.at[pl.ds(sid * STRIPE, STRIPE)])
    plsc.subcore_barrier()

    trip = (NCHUNKS - wid + NW - 1) // NW

    def chunk(t, carry):
        l0 = (wid + t * NW) * CHUNK
        i1 = pltpu.async_copy(head_hbm.at[pl.ds(l0, CHUNK)], headb, insem)
        i2 = pltpu.async_copy(tail_hbm.at[pl.ds(l0, CHUNK)], tailb, insem)
        i3 = pltpu.async_copy(sl_hbm.at[pl.ds(l0, CHUNK)], slb, insem)
        i1.wait()
        i2.wait()
        i3.wait()

        def step(i, c2):
            cs_ = pl.ds(i * 16, 16)
            hh = headb[cs_]
            tt = tailb[cs_]
            slv = slb[cs_]
            bh = plsc.load_gather(base_v, [hh])
            bt = plsc.load_gather(base_v, [tt])
            dvh[cs_] = jnp.where(bt < bh, f32(4097.0), f32(1.0))
            dvt[cs_] = jnp.where(bh < bt, f32(4097.0), f32(1.0))
            svb[cs_] = jnp.abs(slv) * (1.0 / SEC_PER_A)
            dbb[cs_] = bh - bt
            return c2
        lax.fori_loop(0, CHUNK // 16, step, 0)

        o1 = pltpu.async_copy(dbb, dbase_hbm.at[pl.ds(l0, CHUNK)], outsem)
        pltpu.sync_copy(dvh, dsh.at[headb], add=True)
        pltpu.sync_copy(dvt, dsh.at[tailb], add=True)
        pltpu.sync_copy(svb, ssh.at[headb], add=True)
        pltpu.sync_copy(svb, ssh.at[tailb], add=True)
        o1.wait()
        return carry
    lax.fori_loop(0, trip, chunk, 0)

    plsc.subcore_barrier()

    @pl.when(sid == 0)
    def _():
        pltpu.sync_copy(dsh, dpart_hbm.at[cid])
        pltpu.sync_copy(ssh, spart_hbm.at[cid])


_phase1 = pl.kernel(
    _phase1_body,
    out_type=(
        jax.ShapeDtypeStruct((2, NPAD), f32),
        jax.ShapeDtypeStruct((2, NPAD), f32),
        jax.ShapeDtypeStruct((N_LINKS,), f32),
    ),
    mesh=_mesh,
    scratch_types=[
        pltpu.VMEM((NPAD,), f32),
        pltpu.VMEM((CHUNK,), i32),
        pltpu.VMEM((CHUNK,), i32),
        pltpu.VMEM((CHUNK,), f32),
        pltpu.VMEM((CHUNK,), f32),
        pltpu.VMEM((CHUNK,), f32),
        pltpu.VMEM((CHUNK,), f32),
        pltpu.VMEM((CHUNK,), f32),
        pltpu.VMEM((STRIPE,), f32),
        pltpu.VMEM_SHARED((NPAD,), f32),
        pltpu.VMEM_SHARED((NPAD,), f32),
        pltpu.SemaphoreType.DMA,
        pltpu.SemaphoreType.DMA,
    ],
    compiler_params=_sc_params,
)


def _phase2_body(d_ref, s_ref, p_ref, sh_ref, bed_ref, ice_ref, st_ref,
                 open_ref, scl_ref, ptag_ref, bpk_ref):
    D = d_ref[0] + d_ref[1]
    S = s_ref[0] + s_ref[1]
    inds = jnp.floor(D * (1.0 / 4096.0))
    counts = D - 4096.0 * inds
    p = p_ref[...]
    s = sh_ref[...]
    tag = jnp.logical_and(st_ref[...] > 0, D >= 4096.0)
    sliding_node = S / jnp.maximum(counts, 1.0)
    open_ref[...] = jnp.where(
        s < BEDROCK_STEP_HEIGHT,
        sliding_node * (BEDROCK_STEP_HEIGHT - s) * (1.0 / CAVITY_SPACING), 0.0)
    base = f32(WATER_DENSITY * GRAVITY) * bed_ref[...]
    ovb = base + f32(ICE_DENSITY * GRAVITY) * ice_ref[...]
    neff = ovb - p
    rn = jnp.maximum(neff, 0.0)
    scl_ref[...] = f32(CLOSURE_COEFF) * s * (rn * rn * rn)
    pbits = lax.bitcast_convert_type(p, i32)
    ptag_ref[...] = lax.bitcast_convert_type(
        (pbits & jnp.int32(-2)) | tag.astype(i32), f32)
    s16 = lax.bitcast_convert_type(s.astype(jnp.bfloat16), jnp.uint16).astype(i32)
    n16 = lax.bitcast_convert_type(neff.astype(jnp.bfloat16), jnp.uint16).astype(i32)
    bpk_ref[...] = lax.shift_left(n16, 16) | s16


_NSHAPE = (NPAD // 128, 128)


def _phase2(dpart, spart, p2, s2, bed2, ice2, st2):
    return pl.pallas_call(
        _phase2_body,
        out_shape=(
            jax.ShapeDtypeStruct(_NSHAPE, f32),
            jax.ShapeDtypeStruct(_NSHAPE, f32),
            jax.ShapeDtypeStruct(_NSHAPE, f32),
            jax.ShapeDtypeStruct(_NSHAPE, i32),
        ),
    )(dpart, spart, p2, s2, bed2, ice2, st2)


def _phase3_body(head_hbm, tail_hbm, cs_hbm, len_hbm, db_hbm, ptag_hbm, bpk_hbm,
                 diss_hbm, sens_hbm, ccl_hbm,
                 ptag_v, bpk_v, headb, tailb, csb, lenb, dbb, dob, sob, cob,
                 insem, outsem):
    cid = lax.axis_index("c")
    sid = lax.axis_index("s")
    wid = sid * 2 + cid

    pltpu.sync_copy(ptag_hbm, ptag_v)
    pltpu.sync_copy(bpk_hbm, bpk_v)

    trip = (NCHUNKS - wid + NW - 1) // NW

    def chunk(t, carry):
        l0 = (wid + t * NW) * CHUNK
        i1 = pltpu.async_copy(head_hbm.at[pl.ds(l0, CHUNK)], headb, insem)
        i2 = pltpu.async_copy(tail_hbm.at[pl.ds(l0, CHUNK)], tailb, insem)
        i3 = pltpu.async_copy(cs_hbm.at[pl.ds(l0, CHUNK)], csb, insem)
        i4 = pltpu.async_copy(len_hbm.at[pl.ds(l0, CHUNK)], lenb, insem)
        i5 = pltpu.async_copy(db_hbm.at[pl.ds(l0, CHUNK)], dbb, insem)
        i1.wait()
        i2.wait()
        i3.wait()
        i4.wait()
        i5.wait()

        def step(i, c2):
            cs_ = pl.ds(i * 16, 16)
            hh = headb[cs_]
            tt = tailb[cs_]
            ph = plsc.load_gather(ptag_v, [hh])
            pt = plsc.load_gather(ptag_v, [tt])
            bh = plsc.load_gather(bpk_v, [hh])
            bt = plsc.load_gather(bpk_v, [tt])
            phb = plsc.bitcast(ph, i32)
            ptb = plsc.bitcast(pt, i32)
            okm = ((phb | ptb) & 1) == 0
            lenv = lenb[cs_]
            csv = csb[cs_]
            dbv = dbb[cs_]
            rl = 1.0 / lenv
            dp = ph - pt
            g = jnp.where(okm, dp * rl, f32(0.0))
            s_h = plsc.bitcast(lax.shift_left(bh, 16), f32)
            s_t = plsc.bitcast(lax.shift_left(bt, 16), f32)
            ne_h = plsc.bitcast(bh & jnp.int32(-65536), f32)
            ne_t = plsc.bitcast(bt & jnp.int32(-65536), f32)
            h = 0.5 * (s_h + s_t)
            absg = jnp.abs(g)
            rg = _rsqrt2(absg)
            rh = _rsqrt2(h)
            sqh = h * rh
            rq = _rsqrt2(sqh)
            h125 = h * (sqh * rq)
            sheet_q = ((-SHEET_CONDUCTIVITY) * h125 * rg) * g
            chan_q = ((-CHANNEL_CONDUCTIVITY) * (csv * csv * csv)) * g
            dob[cs_] = jnp.abs(CAVITY_SPACING * sheet_q * g) + jnp.abs(chan_q * g)
            pgrad = jnp.where(okm, (dp - dbv) * rl, f32(0.0))
            tq = jnp.where((csv > 0.0) | (pgrad * sheet_q > 0.0),
                           chan_q + CAVITY_SPACING, chan_q)
            sob[cs_] = HEAT_COEFF * tq * pgrad
            nl = jnp.maximum(0.5 * (ne_h + ne_t), 0.0)
            cob[cs_] = f32(CLOSURE_COEFF) * csv * (nl * nl * nl)
            return c2
        lax.fori_loop(0, CHUNK // 16, step, 0)

        o1 = pltpu.async_copy(dob, diss_hbm.at[pl.ds(l0, CHUNK)], outsem)
        o2 = pltpu.async_copy(sob, sens_hbm.at[pl.ds(l0, CHUNK)], outsem)
        o3 = pltpu.async_copy(cob, ccl_hbm.at[pl.ds(l0, CHUNK)], outsem)
        o1.wait()
        o2.wait()
        o3.wait()
        return carry
    lax.fori_loop(0, trip, chunk, 0)


_phase3 = pl.kernel(
    _phase3_body,
    out_type=(
        jax.ShapeDtypeStruct((N_LINKS,), f32),
        jax.ShapeDtypeStruct((N_LINKS,), f32),
        jax.ShapeDtypeStruct((N_LINKS,), f32),
    ),
    mesh=_mesh,
    scratch_types=[
        pltpu.VMEM((NPAD,), f32),
        pltpu.VMEM((NPAD,), i32),
        pltpu.VMEM((CHUNK,), i32),
        pltpu.VMEM((CHUNK,), i32),
        pltpu.VMEM((CHUNK,), f32),
        pltpu.VMEM((CHUNK,), f32),
        pltpu.VMEM((CHUNK,), f32),
        pltpu.VMEM((CHUNK,), f32),
        pltpu.VMEM((CHUNK,), f32),
        pltpu.VMEM((CHUNK,), f32),
        pltpu.SemaphoreType.DMA,
        pltpu.SemaphoreType.DMA,
    ],
    compiler_params=_sc_params,
)


def kernel(potential, sheet_thickness, channel_size, bedrock_elevation,
           ice_thickness, sliding_velocity, link_length,
           node_at_link_head, node_at_link_tail, status_at_node):
    head = node_at_link_head.astype(i32)
    tail = node_at_link_tail.astype(i32)

    pad = NPAD - N_NODES
    bedp = jnp.pad(bedrock_elevation, (0, pad))
    p2 = jnp.pad(potential, (0, pad)).reshape(_NSHAPE)
    s2 = jnp.pad(sheet_thickness, (0, pad)).reshape(_NSHAPE)
    bed2 = bedp.reshape(_NSHAPE)
    ice2 = jnp.pad(ice_thickness, (0, pad)).reshape(_NSHAPE)
    st2 = jnp.pad(status_at_node.astype(i32), (0, pad)).reshape(_NSHAPE)

    dpart, spart, dbase = _phase1(head, tail, sliding_velocity, bedp)

    dpart3 = dpart.reshape(2, NPAD // 128, 128)
    spart3 = spart.reshape(2, NPAD // 128, 128)
    opening, sheet_closure, ptag, bpk = _phase2(
        dpart3, spart3, p2, s2, bed2, ice2, st2)

    diss, sens, ccl = _phase3(
        head, tail, channel_size, link_length, dbase,
        ptag.reshape(NPAD), bpk.reshape(NPAD))

    return (diss, sens,
            opening.reshape(NPAD)[:N_NODES],
            sheet_closure.reshape(NPAD)[:N_NODES],
            ccl)


# speedup vs baseline: 516.7682x; 1.2466x over previous
import jax
import jax.numpy as jnp
from jax import lax
from jax.experimental import pallas as pl
from jax.experimental.pallas import tpu as pltpu
from jax.experimental.pallas import tpu_sc as plsc

N_NODES = 50000
N_LINKS = 1600000
NPAD = 50176
CHUNK = 2000
NCHUNKS = N_LINKS // CHUNK
NW = 32
STRIPE = NPAD // 16

WATER_DENSITY = 1000.0
ICE_DENSITY = 917.0
GRAVITY = 9.81
SEC_PER_A = 31556926.0
SHEET_CONDUCTIVITY = 0.01
CHANNEL_CONDUCTIVITY = 0.1
BEDROCK_STEP_HEIGHT = 0.1
CAVITY_SPACING = 2.0
CLOSURE_COEFF = 5e-25
HEAT_COEFF = -(7.5e-08 * 4220.0 * 1000.0)

f32 = jnp.float32
i32 = jnp.int32

_mesh = plsc.VectorSubcoreMesh(
    core_axis_name="c", subcore_axis_name="s", num_cores=2, num_subcores=16)

_sc_params = pltpu.CompilerParams(needs_layout_passes=False)


def _rsqrt(x):
    bits = plsc.bitcast(x, i32)
    y = plsc.bitcast(jnp.int32(0x5F3759DF) - lax.shift_right_arithmetic(bits, 1), f32)
    for _ in range(3):
        y = y * (1.5 - 0.5 * x * y * y)
    return y


def _rsqrt2(x):
    bits = plsc.bitcast(x, i32)
    y = plsc.bitcast(jnp.int32(0x5F3759DF) - lax.shift_right_arithmetic(bits, 1), f32)
    for _ in range(2):
        y = y * (1.5 - 0.5 * x * y * y)
    return y


def _phase1_body(head_hbm, tail_hbm, sl_hbm, bed_hbm,
                 dpart_hbm, spart_hbm, dbase_hbm,
                 base_v, idx2, slb, val2, sv2, dbb, zb,
                 dsh, ssh, insem, outsem):
    cid = lax.axis_index("c")
    sid = lax.axis_index("s")
    wid = sid * 2 + cid

    pltpu.sync_copy(bed_hbm, base_v)

    def scale(i, carry):
        sl_ = pl.ds(i * 16, 16)
        base_v[sl_] = base_v[sl_] * (WATER_DENSITY * GRAVITY)
        return carry
    lax.fori_loop(0, NPAD // 16, scale, 0)

    def zero(i, carry):
        zb[pl.ds(i * 16, 16)] = jnp.zeros((16,), f32)
        return carry
    lax.fori_loop(0, STRIPE // 16, zero, 0)
    pltpu.sync_copy(zb, dsh.at[pl.ds(sid * STRIPE, STRIPE)])
    pltpu.sync_copy(zb, ssh.at[pl.ds(sid * STRIPE, STRIPE)])
    plsc.subcore_barrier()

    trip = (NCHUNKS - wid + NW - 1) // NW

    def chunk(t, carry):
        l0 = (wid + t * NW) * CHUNK
        i1 = pltpu.async_copy(head_hbm.at[pl.ds(l0, CHUNK)], idx2.at[pl.ds(0, CHUNK)], insem)
        i2 = pltpu.async_copy(tail_hbm.at[pl.ds(l0, CHUNK)], idx2.at[pl.ds(CHUNK, CHUNK)], insem)
        i3 = pltpu.async_copy(sl_hbm.at[pl.ds(l0, CHUNK)], slb, insem)
        i1.wait()
        i2.wait()
        i3.wait()

        @plsc.parallel_loop(0, CHUNK, 16, unroll=4)
        def step(i):
            cs_ = pl.ds(i, 16)
            ct_ = pl.ds(CHUNK + i, 16)
            hh = idx2[cs_]
            tt = idx2[ct_]
            slv = slb[cs_]
            bh = plsc.load_gather(base_v, [hh])
            bt = plsc.load_gather(base_v, [tt])
            val2[cs_] = jnp.where(bt < bh, f32(4097.0), f32(1.0))
            val2[ct_] = jnp.where(bh < bt, f32(4097.0), f32(1.0))
            sv = jnp.abs(slv) * (1.0 / SEC_PER_A)
            sv2[cs_] = sv
            sv2[ct_] = sv
            dbb[cs_] = bh - bt

        o1 = pltpu.async_copy(dbb, dbase_hbm.at[pl.ds(l0, CHUNK)], outsem)
        pltpu.sync_copy(val2, dsh.at[idx2], add=True)
        pltpu.sync_copy(sv2, ssh.at[idx2], add=True)
        o1.wait()
        return carry
    lax.fori_loop(0, trip, chunk, 0)

    plsc.subcore_barrier()

    @pl.when(sid == 0)
    def _():
        pltpu.sync_copy(dsh, dpart_hbm.at[cid])
        pltpu.sync_copy(ssh, spart_hbm.at[cid])


_phase1 = pl.kernel(
    _phase1_body,
    out_type=(
        jax.ShapeDtypeStruct((2, NPAD), f32),
        jax.ShapeDtypeStruct((2, NPAD), f32),
        jax.ShapeDtypeStruct((N_LINKS,), f32),
    ),
    mesh=_mesh,
    scratch_types=[
        pltpu.VMEM((NPAD,), f32),
        pltpu.VMEM((2 * CHUNK,), i32),
        pltpu.VMEM((CHUNK,), f32),
        pltpu.VMEM((2 * CHUNK,), f32),
        pltpu.VMEM((2 * CHUNK,), f32),
        pltpu.VMEM((CHUNK,), f32),
        pltpu.VMEM((STRIPE,), f32),
        pltpu.VMEM_SHARED((NPAD,), f32),
        pltpu.VMEM_SHARED((NPAD,), f32),
        pltpu.SemaphoreType.DMA,
        pltpu.SemaphoreType.DMA,
    ],
    compiler_params=_sc_params,
)


def _phase2_body(d_ref, s_ref, p_ref, sh_ref, bed_ref, ice_ref, st_ref,
                 open_ref, scl_ref, ptag_ref, bpk_ref):
    D = d_ref[0] + d_ref[1]
    S = s_ref[0] + s_ref[1]
    inds = jnp.floor(D * (1.0 / 4096.0))
    counts = D - 4096.0 * inds
    p = p_ref[...]
    s = sh_ref[...]
    tag = jnp.logical_and(st_ref[...] > 0, D >= 4096.0)
    sliding_node = S / jnp.maximum(counts, 1.0)
    open_ref[...] = jnp.where(
        s < BEDROCK_STEP_HEIGHT,
        sliding_node * (BEDROCK_STEP_HEIGHT - s) * (1.0 / CAVITY_SPACING), 0.0)
    base = f32(WATER_DENSITY * GRAVITY) * bed_ref[...]
    ovb = base + f32(ICE_DENSITY * GRAVITY) * ice_ref[...]
    neff = ovb - p
    rn = jnp.maximum(neff, 0.0)
    scl_ref[...] = f32(CLOSURE_COEFF) * s * (rn * rn * rn)
    pbits = lax.bitcast_convert_type(p, i32)
    ptag_ref[...] = lax.bitcast_convert_type(
        (pbits & jnp.int32(-2)) | tag.astype(i32), f32)
    s16 = lax.bitcast_convert_type(s.astype(jnp.bfloat16), jnp.uint16).astype(i32)
    n16 = lax.bitcast_convert_type(neff.astype(jnp.bfloat16), jnp.uint16).astype(i32)
    bpk_ref[...] = lax.shift_left(n16, 16) | s16


_NSHAPE = (NPAD // 128, 128)


def _phase2(dpart, spart, p2, s2, bed2, ice2, st2):
    return pl.pallas_call(
        _phase2_body,
        out_shape=(
            jax.ShapeDtypeStruct(_NSHAPE, f32),
            jax.ShapeDtypeStruct(_NSHAPE, f32),
            jax.ShapeDtypeStruct(_NSHAPE, f32),
            jax.ShapeDtypeStruct(_NSHAPE, i32),
        ),
    )(dpart, spart, p2, s2, bed2, ice2, st2)


def _phase3_body(head_hbm, tail_hbm, cs_hbm, len_hbm, db_hbm, ptag_hbm, bpk_hbm,
                 diss_hbm, sens_hbm, ccl_hbm,
                 ptag_v, bpk_v, headb, tailb, csb, lenb, dbb, dob, sob, cob,
                 insem, outsem):
    cid = lax.axis_index("c")
    sid = lax.axis_index("s")
    wid = sid * 2 + cid

    pltpu.sync_copy(ptag_hbm, ptag_v)
    pltpu.sync_copy(bpk_hbm, bpk_v)

    trip = (NCHUNKS - wid + NW - 1) // NW

    def chunk(t, carry):
        l0 = (wid + t * NW) * CHUNK
        i1 = pltpu.async_copy(head_hbm.at[pl.ds(l0, CHUNK)], headb, insem)
        i2 = pltpu.async_copy(tail_hbm.at[pl.ds(l0, CHUNK)], tailb, insem)
        i3 = pltpu.async_copy(cs_hbm.at[pl.ds(l0, CHUNK)], csb, insem)
        i4 = pltpu.async_copy(len_hbm.at[pl.ds(l0, CHUNK)], lenb, insem)
        i5 = pltpu.async_copy(db_hbm.at[pl.ds(l0, CHUNK)], dbb, insem)
        i1.wait()
        i2.wait()
        i3.wait()
        i4.wait()
        i5.wait()

        @plsc.parallel_loop(0, CHUNK, 16, unroll=4)
        def step(i):
            cs_ = pl.ds(i, 16)
            hh = headb[cs_]
            tt = tailb[cs_]
            ph = plsc.load_gather(ptag_v, [hh])
            pt = plsc.load_gather(ptag_v, [tt])
            bh = plsc.load_gather(bpk_v, [hh])
            bt = plsc.load_gather(bpk_v, [tt])
            phb = plsc.bitcast(ph, i32)
            ptb = plsc.bitcast(pt, i32)
            okm = ((phb | ptb) & 1) == 0
            lenv = lenb[cs_]
            csv = csb[cs_]
            dbv = dbb[cs_]
            rl = 1.0 / lenv
            dp = ph - pt
            g = jnp.where(okm, dp * rl, f32(0.0))
            s_h = plsc.bitcast(lax.shift_left(bh, 16), f32)
            s_t = plsc.bitcast(lax.shift_left(bt, 16), f32)
            ne_h = plsc.bitcast(bh & jnp.int32(-65536), f32)
            ne_t = plsc.bitcast(bt & jnp.int32(-65536), f32)
            h = 0.5 * (s_h + s_t)
            absg = jnp.abs(g)
            rg = _rsqrt2(absg)
            rh = _rsqrt2(h)
            sqh = h * rh
            rq = _rsqrt2(sqh)
            h125 = h * (sqh * rq)
            sheet_q = ((-SHEET_CONDUCTIVITY) * h125 * rg) * g
            chan_q = ((-CHANNEL_CONDUCTIVITY) * (csv * csv * csv)) * g
            dob[cs_] = jnp.abs(CAVITY_SPACING * sheet_q * g) + jnp.abs(chan_q * g)
            pgrad = jnp.where(okm, (dp - dbv) * rl, f32(0.0))
            tq = jnp.where((csv > 0.0) | (pgrad * sheet_q > 0.0),
                           chan_q + CAVITY_SPACING, chan_q)
            sob[cs_] = HEAT_COEFF * tq * pgrad
            nl = jnp.maximum(0.5 * (ne_h + ne_t), 0.0)
            cob[cs_] = f32(CLOSURE_COEFF) * csv * (nl * nl * nl)

        o1 = pltpu.async_copy(dob, diss_hbm.at[pl.ds(l0, CHUNK)], outsem)
        o2 = pltpu.async_copy(sob, sens_hbm.at[pl.ds(l0, CHUNK)], outsem)
        o3 = pltpu.async_copy(cob, ccl_hbm.at[pl.ds(l0, CHUNK)], outsem)
        o1.wait()
        o2.wait()
        o3.wait()
        return carry
    lax.fori_loop(0, trip, chunk, 0)


_phase3 = pl.kernel(
    _phase3_body,
    out_type=(
        jax.ShapeDtypeStruct((N_LINKS,), f32),
        jax.ShapeDtypeStruct((N_LINKS,), f32),
        jax.ShapeDtypeStruct((N_LINKS,), f32),
    ),
    mesh=_mesh,
    scratch_types=[
        pltpu.VMEM((NPAD,), f32),
        pltpu.VMEM((NPAD,), i32),
        pltpu.VMEM((CHUNK,), i32),
        pltpu.VMEM((CHUNK,), i32),
        pltpu.VMEM((CHUNK,), f32),
        pltpu.VMEM((CHUNK,), f32),
        pltpu.VMEM((CHUNK,), f32),
        pltpu.VMEM((CHUNK,), f32),
        pltpu.VMEM((CHUNK,), f32),
        pltpu.VMEM((CHUNK,), f32),
        pltpu.SemaphoreType.DMA,
        pltpu.SemaphoreType.DMA,
    ],
    compiler_params=_sc_params,
)


def kernel(potential, sheet_thickness, channel_size, bedrock_elevation,
           ice_thickness, sliding_velocity, link_length,
           node_at_link_head, node_at_link_tail, status_at_node):
    head = node_at_link_head.astype(i32)
    tail = node_at_link_tail.astype(i32)

    pad = NPAD - N_NODES
    bedp = jnp.pad(bedrock_elevation, (0, pad))
    p2 = jnp.pad(potential, (0, pad)).reshape(_NSHAPE)
    s2 = jnp.pad(sheet_thickness, (0, pad)).reshape(_NSHAPE)
    bed2 = bedp.reshape(_NSHAPE)
    ice2 = jnp.pad(ice_thickness, (0, pad)).reshape(_NSHAPE)
    st2 = jnp.pad(status_at_node.astype(i32), (0, pad)).reshape(_NSHAPE)

    dpart, spart, dbase = _phase1(head, tail, sliding_velocity, bedp)

    dpart3 = dpart.reshape(2, NPAD // 128, 128)
    spart3 = spart.reshape(2, NPAD // 128, 128)
    opening, sheet_closure, ptag, bpk = _phase2(
        dpart3, spart3, p2, s2, bed2, ice2, st2)

    diss, sens, ccl = _phase3(
        head, tail, channel_size, link_length, dbase,
        ptag.reshape(NPAD), bpk.reshape(NPAD))

    return (diss, sens,
            opening.reshape(NPAD)[:N_NODES],
            sheet_closure.reshape(NPAD)[:N_NODES],
            ccl)


# speedup vs baseline: 530.3505x; 1.0263x over previous
import jax
import jax.numpy as jnp
from jax import lax
from jax.experimental import pallas as pl
from jax.experimental.pallas import tpu as pltpu
from jax.experimental.pallas import tpu_sc as plsc

N_NODES = 50000
N_LINKS = 1600000
NPAD = 50176
CHUNK = 2000
NCHUNKS = N_LINKS // CHUNK
NW = 32
STRIPE = NPAD // 16

WATER_DENSITY = 1000.0
ICE_DENSITY = 917.0
GRAVITY = 9.81
SEC_PER_A = 31556926.0
SHEET_CONDUCTIVITY = 0.01
CHANNEL_CONDUCTIVITY = 0.1
BEDROCK_STEP_HEIGHT = 0.1
CAVITY_SPACING = 2.0
CLOSURE_COEFF = 5e-25
HEAT_COEFF = -(7.5e-08 * 4220.0 * 1000.0)

f32 = jnp.float32
i32 = jnp.int32

_mesh = plsc.VectorSubcoreMesh(
    core_axis_name="c", subcore_axis_name="s", num_cores=2, num_subcores=16)

_sc_params = pltpu.CompilerParams(needs_layout_passes=False)


def _rsqrt(x):
    bits = plsc.bitcast(x, i32)
    y = plsc.bitcast(jnp.int32(0x5F3759DF) - lax.shift_right_arithmetic(bits, 1), f32)
    for _ in range(3):
        y = y * (1.5 - 0.5 * x * y * y)
    return y


def _rsqrt2(x):
    bits = plsc.bitcast(x, i32)
    y = plsc.bitcast(jnp.int32(0x5F3759DF) - lax.shift_right_arithmetic(bits, 1), f32)
    for _ in range(2):
        y = y * (1.5 - 0.5 * x * y * y)
    return y


def _phase1_body(head_hbm, tail_hbm, sl_hbm, bed_hbm,
                 dpart_hbm, spart_hbm, dbase_hbm,
                 base_v, idx2, slb, val2, sv2, dbb, zb,
                 dsh, ssh, insem, outsem):
    cid = lax.axis_index("c")
    sid = lax.axis_index("s")
    wid = sid * 2 + cid

    pltpu.sync_copy(bed_hbm, base_v)

    def scale(i, carry):
        sl_ = pl.ds(i * 16, 16)
        base_v[sl_] = base_v[sl_] * (WATER_DENSITY * GRAVITY)
        return carry
    lax.fori_loop(0, NPAD // 16, scale, 0)

    def zero(i, carry):
        zb[pl.ds(i * 16, 16)] = jnp.zeros((16,), f32)
        return carry
    lax.fori_loop(0, STRIPE // 16, zero, 0)
    pltpu.sync_copy(zb, dsh.at[pl.ds(sid * STRIPE, STRIPE)])
    pltpu.sync_copy(zb, ssh.at[pl.ds(sid * STRIPE, STRIPE)])
    plsc.subcore_barrier()

    trip = (NCHUNKS - wid + NW - 1) // NW

    def chunk(t, carry):
        l0 = (wid + t * NW) * CHUNK
        i1 = pltpu.async_copy(head_hbm.at[pl.ds(l0, CHUNK)], idx2.at[pl.ds(0, CHUNK)], insem)
        i2 = pltpu.async_copy(tail_hbm.at[pl.ds(l0, CHUNK)], idx2.at[pl.ds(CHUNK, CHUNK)], insem)
        i3 = pltpu.async_copy(sl_hbm.at[pl.ds(l0, CHUNK)], slb, insem)
        i1.wait()
        i2.wait()
        i3.wait()

        @plsc.parallel_loop(0, CHUNK, 16, unroll=8)
        def step(i):
            cs_ = pl.ds(i, 16)
            ct_ = pl.ds(CHUNK + i, 16)
            hh = idx2[cs_]
            tt = idx2[ct_]
            slv = slb[cs_]
            bh = plsc.load_gather(base_v, [hh])
            bt = plsc.load_gather(base_v, [tt])
            val2[cs_] = jnp.where(bt < bh, f32(4097.0), f32(1.0))
            val2[ct_] = jnp.where(bh < bt, f32(4097.0), f32(1.0))
            sv = jnp.abs(slv) * (1.0 / SEC_PER_A)
            sv2[cs_] = sv
            sv2[ct_] = sv
            dbb[cs_] = bh - bt

        o1 = pltpu.async_copy(dbb, dbase_hbm.at[pl.ds(l0, CHUNK)], outsem)
        pltpu.sync_copy(val2, dsh.at[idx2], add=True)
        pltpu.sync_copy(sv2, ssh.at[idx2], add=True)
        o1.wait()
        return carry
    lax.fori_loop(0, trip, chunk, 0)

    plsc.subcore_barrier()

    @pl.when(sid == 0)
    def _():
        pltpu.sync_copy(dsh, dpart_hbm.at[cid])
        pltpu.sync_copy(ssh, spart_hbm.at[cid])


_phase1 = pl.kernel(
    _phase1_body,
    out_type=(
        jax.ShapeDtypeStruct((2, NPAD), f32),
        jax.ShapeDtypeStruct((2, NPAD), f32),
        jax.ShapeDtypeStruct((N_LINKS,), f32),
    ),
    mesh=_mesh,
    scratch_types=[
        pltpu.VMEM((NPAD,), f32),
        pltpu.VMEM((2 * CHUNK,), i32),
        pltpu.VMEM((CHUNK,), f32),
        pltpu.VMEM((2 * CHUNK,), f32),
        pltpu.VMEM((2 * CHUNK,), f32),
        pltpu.VMEM((CHUNK,), f32),
        pltpu.VMEM((STRIPE,), f32),
        pltpu.VMEM_SHARED((NPAD,), f32),
        pltpu.VMEM_SHARED((NPAD,), f32),
        pltpu.SemaphoreType.DMA,
        pltpu.SemaphoreType.DMA,
    ],
    compiler_params=_sc_params,
)


def _phase2_body(d_ref, s_ref, p_ref, sh_ref, bed_ref, ice_ref, st_ref,
                 open_ref, scl_ref, ptag_ref, bpk_ref):
    D = d_ref[0] + d_ref[1]
    S = s_ref[0] + s_ref[1]
    inds = jnp.floor(D * (1.0 / 4096.0))
    counts = D - 4096.0 * inds
    p = p_ref[...]
    s = sh_ref[...]
    tag = jnp.logical_and(st_ref[...] > 0, D >= 4096.0)
    sliding_node = S / jnp.maximum(counts, 1.0)
    open_ref[...] = jnp.where(
        s < BEDROCK_STEP_HEIGHT,
        sliding_node * (BEDROCK_STEP_HEIGHT - s) * (1.0 / CAVITY_SPACING), 0.0)
    base = f32(WATER_DENSITY * GRAVITY) * bed_ref[...]
    ovb = base + f32(ICE_DENSITY * GRAVITY) * ice_ref[...]
    neff = ovb - p
    rn = jnp.maximum(neff, 0.0)
    scl_ref[...] = f32(CLOSURE_COEFF) * s * (rn * rn * rn)
    pbits = lax.bitcast_convert_type(p, i32)
    ptag_ref[...] = lax.bitcast_convert_type(
        (pbits & jnp.int32(-2)) | tag.astype(i32), f32)
    s16 = lax.bitcast_convert_type(s.astype(jnp.bfloat16), jnp.uint16).astype(i32)
    n16 = lax.bitcast_convert_type(neff.astype(jnp.bfloat16), jnp.uint16).astype(i32)
    bpk_ref[...] = lax.shift_left(n16, 16) | s16


_NSHAPE = (NPAD // 128, 128)


def _phase2(dpart, spart, p2, s2, bed2, ice2, st2):
    return pl.pallas_call(
        _phase2_body,
        out_shape=(
            jax.ShapeDtypeStruct(_NSHAPE, f32),
            jax.ShapeDtypeStruct(_NSHAPE, f32),
            jax.ShapeDtypeStruct(_NSHAPE, f32),
            jax.ShapeDtypeStruct(_NSHAPE, i32),
        ),
    )(dpart, spart, p2, s2, bed2, ice2, st2)


def _phase3_body(head_hbm, tail_hbm, cs_hbm, len_hbm, db_hbm, ptag_hbm, bpk_hbm,
                 diss_hbm, sens_hbm, ccl_hbm,
                 ptag_v, bpk_v, headb, tailb, csb, lenb, dbb, dob, sob, cob,
                 insem, outsem):
    cid = lax.axis_index("c")
    sid = lax.axis_index("s")
    wid = sid * 2 + cid

    pltpu.sync_copy(ptag_hbm, ptag_v)
    pltpu.sync_copy(bpk_hbm, bpk_v)

    trip = (NCHUNKS - wid + NW - 1) // NW

    def chunk(t, carry):
        l0 = (wid + t * NW) * CHUNK
        i1 = pltpu.async_copy(head_hbm.at[pl.ds(l0, CHUNK)], headb, insem)
        i2 = pltpu.async_copy(tail_hbm.at[pl.ds(l0, CHUNK)], tailb, insem)
        i3 = pltpu.async_copy(cs_hbm.at[pl.ds(l0, CHUNK)], csb, insem)
        i4 = pltpu.async_copy(len_hbm.at[pl.ds(l0, CHUNK)], lenb, insem)
        i5 = pltpu.async_copy(db_hbm.at[pl.ds(l0, CHUNK)], dbb, insem)
        i1.wait()
        i2.wait()
        i3.wait()
        i4.wait()
        i5.wait()

        @plsc.parallel_loop(0, CHUNK, 16, unroll=8)
        def step(i):
            cs_ = pl.ds(i, 16)
            hh = headb[cs_]
            tt = tailb[cs_]
            ph = plsc.load_gather(ptag_v, [hh])
            pt = plsc.load_gather(ptag_v, [tt])
            bh = plsc.load_gather(bpk_v, [hh])
            bt = plsc.load_gather(bpk_v, [tt])
            phb = plsc.bitcast(ph, i32)
            ptb = plsc.bitcast(pt, i32)
            okm = ((phb | ptb) & 1) == 0
            lenv = lenb[cs_]
            csv = csb[cs_]
            dbv = dbb[cs_]
            rl = 1.0 / lenv
            dp = ph - pt
            g = jnp.where(okm, dp * rl, f32(0.0))
            s_h = plsc.bitcast(lax.shift_left(bh, 16), f32)
            s_t = plsc.bitcast(lax.shift_left(bt, 16), f32)
            ne_h = plsc.bitcast(bh & jnp.int32(-65536), f32)
            ne_t = plsc.bitcast(bt & jnp.int32(-65536), f32)
            h = 0.5 * (s_h + s_t)
            absg = jnp.abs(g)
            rg = _rsqrt2(absg)
            rh = _rsqrt2(h)
            sqh = h * rh
            rq = _rsqrt2(sqh)
            h125 = h * (sqh * rq)
            sheet_q = ((-SHEET_CONDUCTIVITY) * h125 * rg) * g
            chan_q = ((-CHANNEL_CONDUCTIVITY) * (csv * csv * csv)) * g
            dob[cs_] = jnp.abs(CAVITY_SPACING * sheet_q * g) + jnp.abs(chan_q * g)
            pgrad = jnp.where(okm, (dp - dbv) * rl, f32(0.0))
            tq = jnp.where((csv > 0.0) | (pgrad * sheet_q > 0.0),
                           chan_q + CAVITY_SPACING, chan_q)
            sob[cs_] = HEAT_COEFF * tq * pgrad
            nl = jnp.maximum(0.5 * (ne_h + ne_t), 0.0)
            cob[cs_] = f32(CLOSURE_COEFF) * csv * (nl * nl * nl)

        o1 = pltpu.async_copy(dob, diss_hbm.at[pl.ds(l0, CHUNK)], outsem)
        o2 = pltpu.async_copy(sob, sens_hbm.at[pl.ds(l0, CHUNK)], outsem)
        o3 = pltpu.async_copy(cob, ccl_hbm.at[pl.ds(l0, CHUNK)], outsem)
        o1.wait()
        o2.wait()
        o3.wait()
        return carry
    lax.fori_loop(0, trip, chunk, 0)


_phase3 = pl.kernel(
    _phase3_body,
    out_type=(
        jax.ShapeDtypeStruct((N_LINKS,), f32),
        jax.ShapeDtypeStruct((N_LINKS,), f32),
        jax.ShapeDtypeStruct((N_LINKS,), f32),
    ),
    mesh=_mesh,
    scratch_types=[
        pltpu.VMEM((NPAD,), f32),
        pltpu.VMEM((NPAD,), i32),
        pltpu.VMEM((CHUNK,), i32),
        pltpu.VMEM((CHUNK,), i32),
        pltpu.VMEM((CHUNK,), f32),
        pltpu.VMEM((CHUNK,), f32),
        pltpu.VMEM((CHUNK,), f32),
        pltpu.VMEM((CHUNK,), f32),
        pltpu.VMEM((CHUNK,), f32),
        pltpu.VMEM((CHUNK,), f32),
        pltpu.SemaphoreType.DMA,
        pltpu.SemaphoreType.DMA,
    ],
    compiler_params=_sc_params,
)


def kernel(potential, sheet_thickness, channel_size, bedrock_elevation,
           ice_thickness, sliding_velocity, link_length,
           node_at_link_head, node_at_link_tail, status_at_node):
    head = node_at_link_head.astype(i32)
    tail = node_at_link_tail.astype(i32)

    pad = NPAD - N_NODES
    bedp = jnp.pad(bedrock_elevation, (0, pad))
    p2 = jnp.pad(potential, (0, pad)).reshape(_NSHAPE)
    s2 = jnp.pad(sheet_thickness, (0, pad)).reshape(_NSHAPE)
    bed2 = bedp.reshape(_NSHAPE)
    ice2 = jnp.pad(ice_thickness, (0, pad)).reshape(_NSHAPE)
    st2 = jnp.pad(status_at_node.astype(i32), (0, pad)).reshape(_NSHAPE)

    dpart, spart, dbase = _phase1(head, tail, sliding_velocity, bedp)

    dpart3 = dpart.reshape(2, NPAD // 128, 128)
    spart3 = spart.reshape(2, NPAD // 128, 128)
    opening, sheet_closure, ptag, bpk = _phase2(
        dpart3, spart3, p2, s2, bed2, ice2, st2)

    diss, sens, ccl = _phase3(
        head, tail, channel_size, link_length, dbase,
        ptag.reshape(NPAD), bpk.reshape(NPAD))

    return (diss, sens,
            opening.reshape(NPAD)[:N_NODES],
            sheet_closure.reshape(NPAD)[:N_NODES],
            ccl)


# speedup vs baseline: 531.8081x; 1.0027x over previous
import jax
import jax.numpy as jnp
from jax import lax
from jax.experimental import pallas as pl
from jax.experimental.pallas import tpu as pltpu
from jax.experimental.pallas import tpu_sc as plsc

N_NODES = 50000
N_LINKS = 1600000
NPAD = 50176
CHUNK = 2000
NCHUNKS = N_LINKS // CHUNK
NW = 32
STRIPE = NPAD // 16

WATER_DENSITY = 1000.0
ICE_DENSITY = 917.0
GRAVITY = 9.81
SEC_PER_A = 31556926.0
SHEET_CONDUCTIVITY = 0.01
CHANNEL_CONDUCTIVITY = 0.1
BEDROCK_STEP_HEIGHT = 0.1
CAVITY_SPACING = 2.0
CLOSURE_COEFF = 5e-25
HEAT_COEFF = -(7.5e-08 * 4220.0 * 1000.0)

f32 = jnp.float32
i32 = jnp.int32

_mesh = plsc.VectorSubcoreMesh(
    core_axis_name="c", subcore_axis_name="s", num_cores=2, num_subcores=16)

_sc_params = pltpu.CompilerParams(needs_layout_passes=False)


def _rsqrt(x):
    bits = plsc.bitcast(x, i32)
    y = plsc.bitcast(jnp.int32(0x5F3759DF) - lax.shift_right_arithmetic(bits, 1), f32)
    for _ in range(3):
        y = y * (1.5 - 0.5 * x * y * y)
    return y


def _rsqrt2(x):
    bits = plsc.bitcast(x, i32)
    y = plsc.bitcast(jnp.int32(0x5F3759DF) - lax.shift_right_arithmetic(bits, 1), f32)
    for _ in range(2):
        y = y * (1.5 - 0.5 * x * y * y)
    return y


def _phase1_body(head_hbm, tail_hbm, sl_hbm, bed_hbm,
                 dpart_hbm, spart_hbm, dbase_hbm,
                 base_v, idx2, slb, val2, sv2, dbb, zb,
                 dsh, ssh, insem, outsem, scsem1, scsem2):
    cid = lax.axis_index("c")
    sid = lax.axis_index("s")
    wid = sid * 2 + cid

    pltpu.sync_copy(bed_hbm, base_v)

    def scale(i, carry):
        sl_ = pl.ds(i * 16, 16)
        base_v[sl_] = base_v[sl_] * (WATER_DENSITY * GRAVITY)
        return carry
    lax.fori_loop(0, NPAD // 16, scale, 0)

    def zero(i, carry):
        zb[pl.ds(i * 16, 16)] = jnp.zeros((16,), f32)
        return carry
    lax.fori_loop(0, STRIPE // 16, zero, 0)
    pltpu.sync_copy(zb, dsh.at[pl.ds(sid * STRIPE, STRIPE)])
    pltpu.sync_copy(zb, ssh.at[pl.ds(sid * STRIPE, STRIPE)])
    plsc.subcore_barrier()

    trip = (NCHUNKS - wid + NW - 1) // NW

    def chunk(t, carry):
        l0 = (wid + t * NW) * CHUNK
        i1 = pltpu.async_copy(head_hbm.at[pl.ds(l0, CHUNK)], idx2.at[pl.ds(0, CHUNK)], insem)
        i2 = pltpu.async_copy(tail_hbm.at[pl.ds(l0, CHUNK)], idx2.at[pl.ds(CHUNK, CHUNK)], insem)
        i3 = pltpu.async_copy(sl_hbm.at[pl.ds(l0, CHUNK)], slb, insem)
        i1.wait()
        i2.wait()
        i3.wait()

        @plsc.parallel_loop(0, CHUNK, 16, unroll=8)
        def step(i):
            cs_ = pl.ds(i, 16)
            ct_ = pl.ds(CHUNK + i, 16)
            hh = idx2[cs_]
            tt = idx2[ct_]
            slv = slb[cs_]
            bh = plsc.load_gather(base_v, [hh])
            bt = plsc.load_gather(base_v, [tt])
            val2[cs_] = jnp.where(bt < bh, f32(4097.0), f32(1.0))
            val2[ct_] = jnp.where(bh < bt, f32(4097.0), f32(1.0))
            sv = jnp.abs(slv) * (1.0 / SEC_PER_A)
            sv2[cs_] = sv
            sv2[ct_] = sv
            dbb[cs_] = bh - bt

        o1 = pltpu.async_copy(dbb, dbase_hbm.at[pl.ds(l0, CHUNK)], outsem)
        o2 = pltpu.async_copy(val2, dsh.at[idx2], scsem1, add=True)
        o3 = pltpu.async_copy(sv2, ssh.at[idx2], scsem2, add=True)
        o2.wait()
        o3.wait()
        o1.wait()
        return carry
    lax.fori_loop(0, trip, chunk, 0)

    plsc.subcore_barrier()

    @pl.when(sid == 0)
    def _():
        pltpu.sync_copy(dsh, dpart_hbm.at[cid])
        pltpu.sync_copy(ssh, spart_hbm.at[cid])


_phase1 = pl.kernel(
    _phase1_body,
    out_type=(
        jax.ShapeDtypeStruct((2, NPAD), f32),
        jax.ShapeDtypeStruct((2, NPAD), f32),
        jax.ShapeDtypeStruct((N_LINKS,), f32),
    ),
    mesh=_mesh,
    scratch_types=[
        pltpu.VMEM((NPAD,), f32),
        pltpu.VMEM((2 * CHUNK,), i32),
        pltpu.VMEM((CHUNK,), f32),
        pltpu.VMEM((2 * CHUNK,), f32),
        pltpu.VMEM((2 * CHUNK,), f32),
        pltpu.VMEM((CHUNK,), f32),
        pltpu.VMEM((STRIPE,), f32),
        pltpu.VMEM_SHARED((NPAD,), f32),
        pltpu.VMEM_SHARED((NPAD,), f32),
        pltpu.SemaphoreType.DMA,
        pltpu.SemaphoreType.DMA,
        pltpu.SemaphoreType.DMA,
        pltpu.SemaphoreType.DMA,
    ],
    compiler_params=_sc_params,
)


def _phase2_body(d_ref, s_ref, p_ref, sh_ref, bed_ref, ice_ref, st_ref,
                 open_ref, scl_ref, ptag_ref, bpk_ref):
    D = d_ref[0] + d_ref[1]
    S = s_ref[0] + s_ref[1]
    inds = jnp.floor(D * (1.0 / 4096.0))
    counts = D - 4096.0 * inds
    p = p_ref[...]
    s = sh_ref[...]
    tag = jnp.logical_and(st_ref[...] > 0, D >= 4096.0)
    sliding_node = S / jnp.maximum(counts, 1.0)
    open_ref[...] = jnp.where(
        s < BEDROCK_STEP_HEIGHT,
        sliding_node * (BEDROCK_STEP_HEIGHT - s) * (1.0 / CAVITY_SPACING), 0.0)
    base = f32(WATER_DENSITY * GRAVITY) * bed_ref[...]
    ovb = base + f32(ICE_DENSITY * GRAVITY) * ice_ref[...]
    neff = ovb - p
    rn = jnp.maximum(neff, 0.0)
    scl_ref[...] = f32(CLOSURE_COEFF) * s * (rn * rn * rn)
    pbits = lax.bitcast_convert_type(p, i32)
    ptag_ref[...] = lax.bitcast_convert_type(
        (pbits & jnp.int32(-2)) | tag.astype(i32), f32)
    s16 = lax.bitcast_convert_type(s.astype(jnp.bfloat16), jnp.uint16).astype(i32)
    n16 = lax.bitcast_convert_type(neff.astype(jnp.bfloat16), jnp.uint16).astype(i32)
    bpk_ref[...] = lax.shift_left(n16, 16) | s16


_NSHAPE = (NPAD // 128, 128)


def _phase2(dpart, spart, p2, s2, bed2, ice2, st2):
    return pl.pallas_call(
        _phase2_body,
        out_shape=(
            jax.ShapeDtypeStruct(_NSHAPE, f32),
            jax.ShapeDtypeStruct(_NSHAPE, f32),
            jax.ShapeDtypeStruct(_NSHAPE, f32),
            jax.ShapeDtypeStruct(_NSHAPE, i32),
        ),
    )(dpart, spart, p2, s2, bed2, ice2, st2)


def _phase3_body(head_hbm, tail_hbm, cs_hbm, len_hbm, db_hbm, ptag_hbm, bpk_hbm,
                 diss_hbm, sens_hbm, ccl_hbm,
                 ptag_v, bpk_v, headb, tailb, csb, lenb, dbb, dob, sob, cob,
                 insem, outsem):
    cid = lax.axis_index("c")
    sid = lax.axis_index("s")
    wid = sid * 2 + cid

    pltpu.sync_copy(ptag_hbm, ptag_v)
    pltpu.sync_copy(bpk_hbm, bpk_v)

    trip = (NCHUNKS - wid + NW - 1) // NW

    def chunk(t, carry):
        l0 = (wid + t * NW) * CHUNK
        i1 = pltpu.async_copy(head_hbm.at[pl.ds(l0, CHUNK)], headb, insem)
        i2 = pltpu.async_copy(tail_hbm.at[pl.ds(l0, CHUNK)], tailb, insem)
        i3 = pltpu.async_copy(cs_hbm.at[pl.ds(l0, CHUNK)], csb, insem)
        i4 = pltpu.async_copy(len_hbm.at[pl.ds(l0, CHUNK)], lenb, insem)
        i5 = pltpu.async_copy(db_hbm.at[pl.ds(l0, CHUNK)], dbb, insem)
        i1.wait()
        i2.wait()
        i3.wait()
        i4.wait()
        i5.wait()

        @plsc.parallel_loop(0, CHUNK, 16, unroll=8)
        def step(i):
            cs_ = pl.ds(i, 16)
            hh = headb[cs_]
            tt = tailb[cs_]
            ph = plsc.load_gather(ptag_v, [hh])
            pt = plsc.load_gather(ptag_v, [tt])
            bh = plsc.load_gather(bpk_v, [hh])
            bt = plsc.load_gather(bpk_v, [tt])
            phb = plsc.bitcast(ph, i32)
            ptb = plsc.bitcast(pt, i32)
            okm = ((phb | ptb) & 1) == 0
            lenv = lenb[cs_]
            csv = csb[cs_]
            dbv = dbb[cs_]
            rl = 1.0 / lenv
            dp = ph - pt
            g = jnp.where(okm, dp * rl, f32(0.0))
            s_h = plsc.bitcast(lax.shift_left(bh, 16), f32)
            s_t = plsc.bitcast(lax.shift_left(bt, 16), f32)
            ne_h = plsc.bitcast(bh & jnp.int32(-65536), f32)
            ne_t = plsc.bitcast(bt & jnp.int32(-65536), f32)
            h = 0.5 * (s_h + s_t)
            absg = jnp.abs(g)
            rg = _rsqrt2(absg)
            rh = _rsqrt2(h)
            sqh = h * rh
            rq = _rsqrt2(sqh)
            h125 = h * (sqh * rq)
            sheet_q = ((-SHEET_CONDUCTIVITY) * h125 * rg) * g
            chan_q = ((-CHANNEL_CONDUCTIVITY) * (csv * csv * csv)) * g
            dob[cs_] = jnp.abs(CAVITY_SPACING * sheet_q * g) + jnp.abs(chan_q * g)
            pgrad = jnp.where(okm, (dp - dbv) * rl, f32(0.0))
            tq = jnp.where((csv > 0.0) | (pgrad * sheet_q > 0.0),
                           chan_q + CAVITY_SPACING, chan_q)
            sob[cs_] = HEAT_COEFF * tq * pgrad
            nl = jnp.maximum(0.5 * (ne_h + ne_t), 0.0)
            cob[cs_] = f32(CLOSURE_COEFF) * csv * (nl * nl * nl)

        o1 = pltpu.async_copy(dob, diss_hbm.at[pl.ds(l0, CHUNK)], outsem)
        o2 = pltpu.async_copy(sob, sens_hbm.at[pl.ds(l0, CHUNK)], outsem)
        o3 = pltpu.async_copy(cob, ccl_hbm.at[pl.ds(l0, CHUNK)], outsem)
        o1.wait()
        o2.wait()
        o3.wait()
        return carry
    lax.fori_loop(0, trip, chunk, 0)


_phase3 = pl.kernel(
    _phase3_body,
    out_type=(
        jax.ShapeDtypeStruct((N_LINKS,), f32),
        jax.ShapeDtypeStruct((N_LINKS,), f32),
        jax.ShapeDtypeStruct((N_LINKS,), f32),
    ),
    mesh=_mesh,
    scratch_types=[
        pltpu.VMEM((NPAD,), f32),
        pltpu.VMEM((NPAD,), i32),
        pltpu.VMEM((CHUNK,), i32),
        pltpu.VMEM((CHUNK,), i32),
        pltpu.VMEM((CHUNK,), f32),
        pltpu.VMEM((CHUNK,), f32),
        pltpu.VMEM((CHUNK,), f32),
        pltpu.VMEM((CHUNK,), f32),
        pltpu.VMEM((CHUNK,), f32),
        pltpu.VMEM((CHUNK,), f32),
        pltpu.SemaphoreType.DMA,
        pltpu.SemaphoreType.DMA,
    ],
    compiler_params=_sc_params,
)


def kernel(potential, sheet_thickness, channel_size, bedrock_elevation,
           ice_thickness, sliding_velocity, link_length,
           node_at_link_head, node_at_link_tail, status_at_node):
    head = node_at_link_head.astype(i32)
    tail = node_at_link_tail.astype(i32)

    pad = NPAD - N_NODES
    bedp = jnp.pad(bedrock_elevation, (0, pad))
    p2 = jnp.pad(potential, (0, pad)).reshape(_NSHAPE)
    s2 = jnp.pad(sheet_thickness, (0, pad)).reshape(_NSHAPE)
    bed2 = bedp.reshape(_NSHAPE)
    ice2 = jnp.pad(ice_thickness, (0, pad)).reshape(_NSHAPE)
    st2 = jnp.pad(status_at_node.astype(i32), (0, pad)).reshape(_NSHAPE)

    dpart, spart, dbase = _phase1(head, tail, sliding_velocity, bedp)

    dpart3 = dpart.reshape(2, NPAD // 128, 128)
    spart3 = spart.reshape(2, NPAD // 128, 128)
    opening, sheet_closure, ptag, bpk = _phase2(
        dpart3, spart3, p2, s2, bed2, ice2, st2)

    diss, sens, ccl = _phase3(
        head, tail, channel_size, link_length, dbase,
        ptag.reshape(NPAD), bpk.reshape(NPAD))

    return (diss, sens,
            opening.reshape(NPAD)[:N_NODES],
            sheet_closure.reshape(NPAD)[:N_NODES],
            ccl)


# speedup vs baseline: 605.2592x; 1.1381x over previous
import jax
import jax.numpy as jnp
from jax import lax
from jax.experimental import pallas as pl
from jax.experimental.pallas import tpu as pltpu
from jax.experimental.pallas import tpu_sc as plsc

N_NODES = 50000
N_LINKS = 1600000
NPAD = 50176
CHUNK = 2000
NCHUNKS = N_LINKS // CHUNK
NW = 32
STRIPE = NPAD // 16

WATER_DENSITY = 1000.0
ICE_DENSITY = 917.0
GRAVITY = 9.81
SEC_PER_A = 31556926.0
SHEET_CONDUCTIVITY = 0.01
CHANNEL_CONDUCTIVITY = 0.1
BEDROCK_STEP_HEIGHT = 0.1
CAVITY_SPACING = 2.0
CLOSURE_COEFF = 5e-25
HEAT_COEFF = -(7.5e-08 * 4220.0 * 1000.0)

f32 = jnp.float32
i32 = jnp.int32

_mesh = plsc.VectorSubcoreMesh(
    core_axis_name="c", subcore_axis_name="s", num_cores=2, num_subcores=16)

_sc_params = pltpu.CompilerParams(needs_layout_passes=False)


def _rsqrt(x):
    bits = plsc.bitcast(x, i32)
    y = plsc.bitcast(jnp.int32(0x5F3759DF) - lax.shift_right_arithmetic(bits, 1), f32)
    for _ in range(3):
        y = y * (1.5 - 0.5 * x * y * y)
    return y


def _rsqrt2(x):
    bits = plsc.bitcast(x, i32)
    y = plsc.bitcast(jnp.int32(0x5F3759DF) - lax.shift_right_arithmetic(bits, 1), f32)
    for _ in range(2):
        y = y * (1.5 - 0.5 * x * y * y)
    return y


def _phase1_body(head_hbm, tail_hbm, sl_hbm, bed_hbm,
                 dpart_hbm, spart_hbm, dbase_hbm,
                 base_v, idx2a, slba, val2a, sv2a, dbba,
                 idx2b, slbb, val2b, sv2b, dbbb, zb,
                 dsh, ssh, insem, outsem, scsem1, scsem2):
    cid = lax.axis_index("c")
    sid = lax.axis_index("s")
    wid = sid * 2 + cid

    pltpu.sync_copy(bed_hbm, base_v)

    def scale(i, carry):
        sl_ = pl.ds(i * 16, 16)
        base_v[sl_] = base_v[sl_] * (WATER_DENSITY * GRAVITY)
        return carry
    lax.fori_loop(0, NPAD // 16, scale, 0)

    def zero(i, carry):
        zb[pl.ds(i * 16, 16)] = jnp.zeros((16,), f32)
        return carry
    lax.fori_loop(0, STRIPE // 16, zero, 0)
    pltpu.sync_copy(zb, dsh.at[pl.ds(sid * STRIPE, STRIPE)])
    pltpu.sync_copy(zb, ssh.at[pl.ds(sid * STRIPE, STRIPE)])
    plsc.subcore_barrier()

    trip = (NCHUNKS - wid + NW - 1) // NW

    def issue_inputs(t, idx2, slb):
        l0 = (wid + t * NW) * CHUNK
        pltpu.async_copy(head_hbm.at[pl.ds(l0, CHUNK)], idx2.at[pl.ds(0, CHUNK)], insem)
        pltpu.async_copy(tail_hbm.at[pl.ds(l0, CHUNK)], idx2.at[pl.ds(CHUNK, CHUNK)], insem)
        pltpu.async_copy(sl_hbm.at[pl.ds(l0, CHUNK)], slb, insem)

    def drain_inputs(idx2, slb):
        pltpu.make_async_copy(head_hbm.at[pl.ds(0, CHUNK)], idx2.at[pl.ds(0, CHUNK)], insem).wait()
        pltpu.make_async_copy(tail_hbm.at[pl.ds(0, CHUNK)], idx2.at[pl.ds(CHUNK, CHUNK)], insem).wait()
        pltpu.make_async_copy(sl_hbm.at[pl.ds(0, CHUNK)], slb, insem).wait()

    def drain_outputs(idx2, val2, sv2, dbb):
        pltpu.make_async_copy(dbb, dbase_hbm.at[pl.ds(0, CHUNK)], outsem).wait()
        pltpu.make_async_copy(val2, dsh.at[idx2], scsem1).wait()
        pltpu.make_async_copy(sv2, ssh.at[idx2], scsem2).wait()

    def do_iter(t, cur, oth):
        idx2, slb, val2, sv2, dbb = cur
        l0 = (wid + t * NW) * CHUNK
        drain_inputs(idx2, slb)

        @plsc.parallel_loop(0, CHUNK, 16, unroll=8)
        def step(i):
            cs_ = pl.ds(i, 16)
            ct_ = pl.ds(CHUNK + i, 16)
            hh = idx2[cs_]
            tt = idx2[ct_]
            slv = slb[cs_]
            bh = plsc.load_gather(base_v, [hh])
            bt = plsc.load_gather(base_v, [tt])
            val2[cs_] = jnp.where(bt < bh, f32(4097.0), f32(1.0))
            val2[ct_] = jnp.where(bh < bt, f32(4097.0), f32(1.0))
            sv = jnp.abs(slv) * (1.0 / SEC_PER_A)
            sv2[cs_] = sv
            sv2[ct_] = sv
            dbb[cs_] = bh - bt

        @pl.when(t >= 1)
        def _():
            drain_outputs(oth[0], oth[2], oth[3], oth[4])
        pltpu.async_copy(dbb, dbase_hbm.at[pl.ds(l0, CHUNK)], outsem)
        pltpu.async_copy(val2, dsh.at[idx2], scsem1, add=True)
        pltpu.async_copy(sv2, ssh.at[idx2], scsem2, add=True)

        @pl.when(t + 1 < trip)
        def _():
            issue_inputs(t + 1, oth[0], oth[1])

    bufa = (idx2a, slba, val2a, sv2a, dbba)
    bufb = (idx2b, slbb, val2b, sv2b, dbbb)

    issue_inputs(0, idx2a, slba)

    def chunk(t, carry):
        even = lax.rem(t, 2) == 0

        @pl.when(even)
        def _():
            do_iter(t, bufa, bufb)

        @pl.when(jnp.logical_not(even))
        def _():
            do_iter(t, bufb, bufa)
        return carry
    lax.fori_loop(0, trip, chunk, 0)

    @pl.when(lax.rem(trip - 1, 2) == 0)
    def _():
        drain_outputs(idx2a, val2a, sv2a, dbba)

    @pl.when(lax.rem(trip - 1, 2) == 1)
    def _():
        drain_outputs(idx2b, val2b, sv2b, dbbb)

    plsc.subcore_barrier()

    @pl.when(sid == 0)
    def _():
        pltpu.sync_copy(dsh, dpart_hbm.at[cid])
        pltpu.sync_copy(ssh, spart_hbm.at[cid])


_phase1 = pl.kernel(
    _phase1_body,
    out_type=(
        jax.ShapeDtypeStruct((2, NPAD), f32),
        jax.ShapeDtypeStruct((2, NPAD), f32),
        jax.ShapeDtypeStruct((N_LINKS,), f32),
    ),
    mesh=_mesh,
    scratch_types=[
        pltpu.VMEM((NPAD,), f32),
        pltpu.VMEM((2 * CHUNK,), i32),
        pltpu.VMEM((CHUNK,), f32),
        pltpu.VMEM((2 * CHUNK,), f32),
        pltpu.VMEM((2 * CHUNK,), f32),
        pltpu.VMEM((CHUNK,), f32),
        pltpu.VMEM((2 * CHUNK,), i32),
        pltpu.VMEM((CHUNK,), f32),
        pltpu.VMEM((2 * CHUNK,), f32),
        pltpu.VMEM((2 * CHUNK,), f32),
        pltpu.VMEM((CHUNK,), f32),
        pltpu.VMEM((STRIPE,), f32),
        pltpu.VMEM_SHARED((NPAD,), f32),
        pltpu.VMEM_SHARED((NPAD,), f32),
        pltpu.SemaphoreType.DMA,
        pltpu.SemaphoreType.DMA,
        pltpu.SemaphoreType.DMA,
        pltpu.SemaphoreType.DMA,
    ],
    compiler_params=_sc_params,
)


def _phase2_body(d_ref, s_ref, p_ref, sh_ref, bed_ref, ice_ref, st_ref,
                 open_ref, scl_ref, ptag_ref, bpk_ref):
    D = d_ref[0] + d_ref[1]
    S = s_ref[0] + s_ref[1]
    inds = jnp.floor(D * (1.0 / 4096.0))
    counts = D - 4096.0 * inds
    p = p_ref[...]
    s = sh_ref[...]
    tag = jnp.logical_and(st_ref[...] > 0, D >= 4096.0)
    sliding_node = S / jnp.maximum(counts, 1.0)
    open_ref[...] = jnp.where(
        s < BEDROCK_STEP_HEIGHT,
        sliding_node * (BEDROCK_STEP_HEIGHT - s) * (1.0 / CAVITY_SPACING), 0.0)
    base = f32(WATER_DENSITY * GRAVITY) * bed_ref[...]
    ovb = base + f32(ICE_DENSITY * GRAVITY) * ice_ref[...]
    neff = ovb - p
    rn = jnp.maximum(neff, 0.0)
    scl_ref[...] = f32(CLOSURE_COEFF) * s * (rn * rn * rn)
    pbits = lax.bitcast_convert_type(p, i32)
    ptag_ref[...] = lax.bitcast_convert_type(
        (pbits & jnp.int32(-2)) | tag.astype(i32), f32)
    s16 = lax.bitcast_convert_type(s.astype(jnp.bfloat16), jnp.uint16).astype(i32)
    n16 = lax.bitcast_convert_type(neff.astype(jnp.bfloat16), jnp.uint16).astype(i32)
    bpk_ref[...] = lax.shift_left(n16, 16) | s16


_NSHAPE = (NPAD // 128, 128)


def _phase2(dpart, spart, p2, s2, bed2, ice2, st2):
    return pl.pallas_call(
        _phase2_body,
        out_shape=(
            jax.ShapeDtypeStruct(_NSHAPE, f32),
            jax.ShapeDtypeStruct(_NSHAPE, f32),
            jax.ShapeDtypeStruct(_NSHAPE, f32),
            jax.ShapeDtypeStruct(_NSHAPE, i32),
        ),
    )(dpart, spart, p2, s2, bed2, ice2, st2)


def _phase3_body(head_hbm, tail_hbm, cs_hbm, len_hbm, db_hbm, ptag_hbm, bpk_hbm,
                 diss_hbm, sens_hbm, ccl_hbm,
                 ptag_v, bpk_v, headb, tailb, csb, lenb, dbb, dob, sob, cob,
                 insem, outsem):
    cid = lax.axis_index("c")
    sid = lax.axis_index("s")
    wid = sid * 2 + cid

    pltpu.sync_copy(ptag_hbm, ptag_v)
    pltpu.sync_copy(bpk_hbm, bpk_v)

    trip = (NCHUNKS - wid + NW - 1) // NW

    def chunk(t, carry):
        l0 = (wid + t * NW) * CHUNK
        i1 = pltpu.async_copy(head_hbm.at[pl.ds(l0, CHUNK)], headb, insem)
        i2 = pltpu.async_copy(tail_hbm.at[pl.ds(l0, CHUNK)], tailb, insem)
        i3 = pltpu.async_copy(cs_hbm.at[pl.ds(l0, CHUNK)], csb, insem)
        i4 = pltpu.async_copy(len_hbm.at[pl.ds(l0, CHUNK)], lenb, insem)
        i5 = pltpu.async_copy(db_hbm.at[pl.ds(l0, CHUNK)], dbb, insem)
        i1.wait()
        i2.wait()
        i3.wait()
        i4.wait()
        i5.wait()

        @plsc.parallel_loop(0, CHUNK, 16, unroll=8)
        def step(i):
            cs_ = pl.ds(i, 16)
            hh = headb[cs_]
            tt = tailb[cs_]
            ph = plsc.load_gather(ptag_v, [hh])
            pt = plsc.load_gather(ptag_v, [tt])
            bh = plsc.load_gather(bpk_v, [hh])
            bt = plsc.load_gather(bpk_v, [tt])
            phb = plsc.bitcast(ph, i32)
            ptb = plsc.bitcast(pt, i32)
            okm = ((phb | ptb) & 1) == 0
            lenv = lenb[cs_]
            csv = csb[cs_]
            dbv = dbb[cs_]
            rl = 1.0 / lenv
            dp = ph - pt
            g = jnp.where(okm, dp * rl, f32(0.0))
            s_h = plsc.bitcast(lax.shift_left(bh, 16), f32)
            s_t = plsc.bitcast(lax.shift_left(bt, 16), f32)
            ne_h = plsc.bitcast(bh & jnp.int32(-65536), f32)
            ne_t = plsc.bitcast(bt & jnp.int32(-65536), f32)
            h = 0.5 * (s_h + s_t)
            absg = jnp.abs(g)
            rg = _rsqrt2(absg)
            rh = _rsqrt2(h)
            sqh = h * rh
            rq = _rsqrt2(sqh)
            h125 = h * (sqh * rq)
            sheet_q = ((-SHEET_CONDUCTIVITY) * h125 * rg) * g
            chan_q = ((-CHANNEL_CONDUCTIVITY) * (csv * csv * csv)) * g
            dob[cs_] = jnp.abs(CAVITY_SPACING * sheet_q * g) + jnp.abs(chan_q * g)
            pgrad = jnp.where(okm, (dp - dbv) * rl, f32(0.0))
            tq = jnp.where((csv > 0.0) | (pgrad * sheet_q > 0.0),
                           chan_q + CAVITY_SPACING, chan_q)
            sob[cs_] = HEAT_COEFF * tq * pgrad
            nl = jnp.maximum(0.5 * (ne_h + ne_t), 0.0)
            cob[cs_] = f32(CLOSURE_COEFF) * csv * (nl * nl * nl)

        o1 = pltpu.async_copy(dob, diss_hbm.at[pl.ds(l0, CHUNK)], outsem)
        o2 = pltpu.async_copy(sob, sens_hbm.at[pl.ds(l0, CHUNK)], outsem)
        o3 = pltpu.async_copy(cob, ccl_hbm.at[pl.ds(l0, CHUNK)], outsem)
        o1.wait()
        o2.wait()
        o3.wait()
        return carry
    lax.fori_loop(0, trip, chunk, 0)


_phase3 = pl.kernel(
    _phase3_body,
    out_type=(
        jax.ShapeDtypeStruct((N_LINKS,), f32),
        jax.ShapeDtypeStruct((N_LINKS,), f32),
        jax.ShapeDtypeStruct((N_LINKS,), f32),
    ),
    mesh=_mesh,
    scratch_types=[
        pltpu.VMEM((NPAD,), f32),
        pltpu.VMEM((NPAD,), i32),
        pltpu.VMEM((CHUNK,), i32),
        pltpu.VMEM((CHUNK,), i32),
        pltpu.VMEM((CHUNK,), f32),
        pltpu.VMEM((CHUNK,), f32),
        pltpu.VMEM((CHUNK,), f32),
        pltpu.VMEM((CHUNK,), f32),
        pltpu.VMEM((CHUNK,), f32),
        pltpu.VMEM((CHUNK,), f32),
        pltpu.SemaphoreType.DMA,
        pltpu.SemaphoreType.DMA,
    ],
    compiler_params=_sc_params,
)


def kernel(potential, sheet_thickness, channel_size, bedrock_elevation,
           ice_thickness, sliding_velocity, link_length,
           node_at_link_head, node_at_link_tail, status_at_node):
    head = node_at_link_head.astype(i32)
    tail = node_at_link_tail.astype(i32)

    pad = NPAD - N_NODES
    bedp = jnp.pad(bedrock_elevation, (0, pad))
    p2 = jnp.pad(potential, (0, pad)).reshape(_NSHAPE)
    s2 = jnp.pad(sheet_thickness, (0, pad)).reshape(_NSHAPE)
    bed2 = bedp.reshape(_NSHAPE)
    ice2 = jnp.pad(ice_thickness, (0, pad)).reshape(_NSHAPE)
    st2 = jnp.pad(status_at_node.astype(i32), (0, pad)).reshape(_NSHAPE)

    dpart, spart, dbase = _phase1(head, tail, sliding_velocity, bedp)

    dpart3 = dpart.reshape(2, NPAD // 128, 128)
    spart3 = spart.reshape(2, NPAD // 128, 128)
    opening, sheet_closure, ptag, bpk = _phase2(
        dpart3, spart3, p2, s2, bed2, ice2, st2)

    diss, sens, ccl = _phase3(
        head, tail, channel_size, link_length, dbase,
        ptag.reshape(NPAD), bpk.reshape(NPAD))

    return (diss, sens,
            opening.reshape(NPAD)[:N_NODES],
            sheet_closure.reshape(NPAD)[:N_NODES],
            ccl)
